# Initial kernel scaffold; baseline (speedup 1.0000x reference)
import jax
import jax.numpy as jnp
from jax import lax
from jax.experimental import pallas as pl
from jax.experimental.pallas import tpu as pltpu
from jax.experimental.pallas import tpu_sc as plsc

N_ATOMS = 150000
N_STRUCT = 1000
N_FEAT = 128
N_SPECIES = 4

NC = 2
NS = 16

CHUNK = 125
CHUNKS_PER_TILE = N_ATOMS // (NS * CHUNK)
ROWS_PER_TILE = CHUNK * CHUNKS_PER_TILE
ZROWS = 2 * N_STRUCT // NS


def _sc_body(f0, i0, f1, i1, f2, i2, f3, i3, out_hbm,
             acc_a, acc_b, zbuf, idxbuf, fbuf):
  cid = lax.axis_index("c")
  sid = lax.axis_index("s")

  def zstore(i, _):
    r = i // (N_FEAT // 16)
    c = (i % (N_FEAT // 16)) * 16
    zbuf[r, pl.ds(c, 16)] = jnp.zeros((16,), jnp.float32)
    return _
  lax.fori_loop(0, ZROWS * (N_FEAT // 16), zstore, None)

  @pl.when(sid < NS // 2)
  def _():
    pltpu.sync_copy(zbuf, acc_a.at[pl.ds(sid * ZROWS, ZROWS)])

  @pl.when(sid >= NS // 2)
  def _():
    pltpu.sync_copy(zbuf, acc_b.at[pl.ds((sid - NS // 2) * ZROWS, ZROWS)])

  plsc.subcore_barrier()

  def process(f_hbm, idx_hbm, acc):
    pltpu.sync_copy(idx_hbm.at[sid], idxbuf)

    def body(k, carry):
      pltpu.sync_copy(
          f_hbm.at[pl.ds(sid * ROWS_PER_TILE + k * CHUNK, CHUNK)], fbuf)
      pltpu.sync_copy(fbuf, acc.at[idxbuf.at[k]], add=True)
      return carry

    lax.fori_loop(0, CHUNKS_PER_TILE, body, None)

  @pl.when(cid == 0)
  def _():
    process(f0, i0, acc_a)
    process(f1, i1, acc_b)

  @pl.when(cid == 1)
  def _():
    process(f2, i2, acc_a)
    process(f3, i3, acc_b)

  plsc.subcore_barrier()

  @pl.when(sid < NS // 2)
  def _():
    pltpu.sync_copy(acc_a.at[pl.ds(sid * ZROWS, ZROWS)],
                    out_hbm.at[2 * cid, pl.ds(sid * ZROWS, ZROWS)])

  @pl.when(sid >= NS // 2)
  def _():
    pltpu.sync_copy(acc_b.at[pl.ds((sid - NS // 2) * ZROWS, ZROWS)],
                    out_hbm.at[2 * cid + 1, pl.ds((sid - NS // 2) * ZROWS, ZROWS)])


@jax.jit
def _sum_atoms(f0, i0, f1, i1, f2, i2, f3, i3):
  mesh = plsc.VectorSubcoreMesh(
      core_axis_name="c", subcore_axis_name="s", num_cores=NC, num_subcores=NS)
  call = pl.kernel(
      _sc_body,
      out_type=jax.ShapeDtypeStruct((N_SPECIES, N_STRUCT, N_FEAT), jnp.float32),
      mesh=mesh,
      scratch_types=[
          pltpu.VMEM_SHARED((N_STRUCT, N_FEAT), jnp.float32),
          pltpu.VMEM_SHARED((N_STRUCT, N_FEAT), jnp.float32),
          pltpu.VMEM((ZROWS, N_FEAT), jnp.float32),
          pltpu.VMEM((CHUNKS_PER_TILE, CHUNK), jnp.int32),
          pltpu.VMEM((CHUNK, N_FEAT), jnp.float32),
      ],
  )
  return call(f0, i0, f1, i1, f2, i2, f3, i3)


def kernel(features_1, structure_indices_1, features_6, structure_indices_6,
           features_7, structure_indices_7, features_8, structure_indices_8):
  def prep(idx):
    return idx.astype(jnp.int32).reshape(NS, CHUNKS_PER_TILE, CHUNK)

  out = _sum_atoms(
      features_1, prep(structure_indices_1),
      features_6, prep(structure_indices_6),
      features_7, prep(structure_indices_7),
      features_8, prep(structure_indices_8))
  return out.transpose(1, 0, 2).reshape(N_STRUCT, N_SPECIES * N_FEAT)


# speedup vs baseline: 8.6148x; 8.6148x over previous
import jax
import jax.numpy as jnp
from jax import lax
from jax.experimental import pallas as pl
from jax.experimental.pallas import tpu as pltpu
from jax.experimental.pallas import tpu_sc as plsc

N_ATOMS = 150000
N_STRUCT = 1000
N_FEAT = 128
N_SPECIES = 4

NC = 2
NS = 16

CHUNK = 125
CHUNKS_PER_TILE = N_ATOMS // (NS * CHUNK)
ROWS_PER_TILE = CHUNK * CHUNKS_PER_TILE
ZROWS = 2 * N_STRUCT // NS


def _sc_body(f0, i0, f1, i1, f2, i2, f3, i3, out_hbm,
             acc_a, acc_b, zbuf, idxbuf, fbuf):
  cid = lax.axis_index("c")
  sid = lax.axis_index("s")

  def zstore(i, _):
    r = i // (N_FEAT // 16)
    c = (i % (N_FEAT // 16)) * 16
    zbuf[r, pl.ds(c, 16)] = jnp.zeros((16,), jnp.float32)
    return _
  lax.fori_loop(0, ZROWS * (N_FEAT // 16), zstore, None)

  @pl.when(sid < NS // 2)
  def _():
    pltpu.sync_copy(zbuf, acc_a.at[pl.ds(sid * ZROWS, ZROWS)])

  @pl.when(sid >= NS // 2)
  def _():
    pltpu.sync_copy(zbuf, acc_b.at[pl.ds((sid - NS // 2) * ZROWS, ZROWS)])

  plsc.subcore_barrier()

  def process(f_hbm, idx_hbm, acc):
    pltpu.sync_copy(idx_hbm.at[sid], idxbuf)

    def body(k, carry):
      pltpu.sync_copy(
          f_hbm.at[pl.ds(sid * ROWS_PER_TILE + k * CHUNK, CHUNK)], fbuf)
      pltpu.sync_copy(fbuf, acc.at[idxbuf.at[k]], add=True)
      return carry

    lax.fori_loop(0, CHUNKS_PER_TILE, body, None)

  @pl.when(cid == 0)
  def _():
    process(f0, i0, acc_a)
    process(f1, i1, acc_b)

  @pl.when(cid == 1)
  def _():
    process(f2, i2, acc_a)
    process(f3, i3, acc_b)

  plsc.subcore_barrier()

  @pl.when(sid < NS // 2)
  def _():
    pltpu.sync_copy(acc_a.at[pl.ds(sid * ZROWS, ZROWS)],
                    out_hbm.at[2 * cid, pl.ds(sid * ZROWS, ZROWS)])

  @pl.when(sid >= NS // 2)
  def _():
    pltpu.sync_copy(acc_b.at[pl.ds((sid - NS // 2) * ZROWS, ZROWS)],
                    out_hbm.at[2 * cid + 1, pl.ds((sid - NS // 2) * ZROWS, ZROWS)])


@jax.jit
def _sum_atoms(f0, i0, f1, i1, f2, i2, f3, i3):
  mesh = plsc.VectorSubcoreMesh(
      core_axis_name="c", subcore_axis_name="s", num_cores=NC, num_subcores=NS)
  call = pl.kernel(
      _sc_body,
      out_type=jax.ShapeDtypeStruct((N_SPECIES, N_STRUCT, N_FEAT), jnp.float32),
      mesh=mesh,
      scratch_types=[
          pltpu.VMEM_SHARED((N_STRUCT, N_FEAT), jnp.float32),
          pltpu.VMEM_SHARED((N_STRUCT, N_FEAT), jnp.float32),
          pltpu.VMEM((ZROWS, N_FEAT), jnp.float32),
          pltpu.VMEM((CHUNKS_PER_TILE, CHUNK), jnp.int32),
          pltpu.VMEM((CHUNK, N_FEAT), jnp.float32),
      ],
      compiler_params=pltpu.CompilerParams(use_tc_tiling_on_sc=False),
  )
  return call(f0, i0, f1, i1, f2, i2, f3, i3)


def kernel(features_1, structure_indices_1, features_6, structure_indices_6,
           features_7, structure_indices_7, features_8, structure_indices_8):
  def prep(idx):
    return idx.astype(jnp.int32).reshape(NS, CHUNKS_PER_TILE, CHUNK)

  out = _sum_atoms(
      features_1, prep(structure_indices_1),
      features_6, prep(structure_indices_6),
      features_7, prep(structure_indices_7),
      features_8, prep(structure_indices_8))
  return out.transpose(1, 0, 2).reshape(N_STRUCT, N_SPECIES * N_FEAT)


# speedup vs baseline: 14.5830x; 1.6928x over previous
import jax
import jax.numpy as jnp
from jax import lax
from jax.experimental import pallas as pl
from jax.experimental.pallas import tpu as pltpu
from jax.experimental.pallas import tpu_sc as plsc

N_ATOMS = 150000
N_STRUCT = 1000
N_FEAT = 128
N_SPECIES = 4

NC = 2
NS = 16

CHUNK = 125
CHUNKS_PER_TILE = N_ATOMS // (NS * CHUNK)
ROWS_PER_TILE = CHUNK * CHUNKS_PER_TILE
ZROWS = 2 * N_STRUCT // NS
NBUF = 3
NGROUPS = CHUNKS_PER_TILE // NBUF


def _sc_body(f0, i0, f1, i1, f2, i2, f3, i3, out_hbm,
             acc_a, acc_b, zbuf, idxbuf, fbuf, lsem):
  cid = lax.axis_index("c")
  sid = lax.axis_index("s")

  def zstore(i, _):
    r = i // (N_FEAT // 16)
    c = (i % (N_FEAT // 16)) * 16
    zbuf[r, pl.ds(c, 16)] = jnp.zeros((16,), jnp.float32)
    return _
  lax.fori_loop(0, ZROWS * (N_FEAT // 16), zstore, None)

  @pl.when(sid < NS // 2)
  def _():
    pltpu.sync_copy(zbuf, acc_a.at[pl.ds(sid * ZROWS, ZROWS)])

  @pl.when(sid >= NS // 2)
  def _():
    pltpu.sync_copy(zbuf, acc_b.at[pl.ds((sid - NS // 2) * ZROWS, ZROWS)])

  plsc.subcore_barrier()

  def process(f_hbm, idx_hbm, acc):
    pltpu.sync_copy(idx_hbm.at[sid], idxbuf)

    def src(k):
      return f_hbm.at[pl.ds(sid * ROWS_PER_TILE + k * CHUNK, CHUNK)]

    for b in range(NBUF):
      pltpu.async_copy(src(b), fbuf.at[b], lsem.at[b])

    def group(g, carry):
      for b in range(NBUF):
        k = g * NBUF + b
        pltpu.make_async_copy(src(k), fbuf.at[b], lsem.at[b]).wait()
        pltpu.sync_copy(fbuf.at[b], acc.at[idxbuf.at[k]], add=True)

        @pl.when(g < NGROUPS - 1)
        def _():
          pltpu.async_copy(src(k + NBUF), fbuf.at[b], lsem.at[b])
      return carry

    lax.fori_loop(0, NGROUPS, group, None)

  @pl.when(cid == 0)
  def _():
    process(f0, i0, acc_a)
    process(f1, i1, acc_b)

  @pl.when(cid == 1)
  def _():
    process(f2, i2, acc_a)
    process(f3, i3, acc_b)

  plsc.subcore_barrier()

  @pl.when(sid < NS // 2)
  def _():
    pltpu.sync_copy(acc_a.at[pl.ds(sid * ZROWS, ZROWS)],
                    out_hbm.at[2 * cid, pl.ds(sid * ZROWS, ZROWS)])

  @pl.when(sid >= NS // 2)
  def _():
    pltpu.sync_copy(acc_b.at[pl.ds((sid - NS // 2) * ZROWS, ZROWS)],
                    out_hbm.at[2 * cid + 1, pl.ds((sid - NS // 2) * ZROWS, ZROWS)])


@jax.jit
def _sum_atoms(f0, i0, f1, i1, f2, i2, f3, i3):
  mesh = plsc.VectorSubcoreMesh(
      core_axis_name="c", subcore_axis_name="s", num_cores=NC, num_subcores=NS)
  call = pl.kernel(
      _sc_body,
      out_type=jax.ShapeDtypeStruct((N_SPECIES, N_STRUCT, N_FEAT), jnp.float32),
      mesh=mesh,
      scratch_types=[
          pltpu.VMEM_SHARED((N_STRUCT, N_FEAT), jnp.float32),
          pltpu.VMEM_SHARED((N_STRUCT, N_FEAT), jnp.float32),
          pltpu.VMEM((ZROWS, N_FEAT), jnp.float32),
          pltpu.VMEM((CHUNKS_PER_TILE, CHUNK), jnp.int32),
          pltpu.VMEM((NBUF, CHUNK, N_FEAT), jnp.float32),
          pltpu.SemaphoreType.DMA((NBUF,)),
      ],
      compiler_params=pltpu.CompilerParams(use_tc_tiling_on_sc=False),
  )
  return call(f0, i0, f1, i1, f2, i2, f3, i3)


def kernel(features_1, structure_indices_1, features_6, structure_indices_6,
           features_7, structure_indices_7, features_8, structure_indices_8):
  def prep(idx):
    return idx.astype(jnp.int32).reshape(NS, CHUNKS_PER_TILE, CHUNK)

  out = _sum_atoms(
      features_1, prep(structure_indices_1),
      features_6, prep(structure_indices_6),
      features_7, prep(structure_indices_7),
      features_8, prep(structure_indices_8))
  return out.transpose(1, 0, 2).reshape(N_STRUCT, N_SPECIES * N_FEAT)
